# baseline (speedup 1.0000x reference)
import functools
import math

import jax
import jax.numpy as jnp
from jax import lax
from jax.experimental import pallas as pl
from jax.experimental.pallas import tpu as pltpu
from jax.experimental.pallas import tpu_sc as plsc

_VOCAB = 1000
_D = 128
_B, _N = 64, 32
_K = 100
_BN = _B * _N
_NPAD = 104
_LOGK = math.log(_K)

_NC, _NS = 2, 16
_NW = _NC * _NS
_TPW = _BN // _NW
_NPW = 8


def _sc_gather_body(tgt_hbm, noise_hbm, emb_hbm, rows_out,
                    idx_v, rows_v, nidx_v, nrows_v, sem, nsem, wsem):
    wid = lax.axis_index("s") * _NC + lax.axis_index("c")
    base = wid * _TPW
    nrow = _TPW // _N
    pltpu.sync_copy(tgt_hbm.at[pl.ds(wid * nrow, nrow), :], idx_v)
    for r in range(nrow):
        pltpu.async_copy(emb_hbm.at[idx_v.at[r]],
                         rows_v.at[pl.ds(r * _N, _N)], sem)

    nfull = _K // _NPW
    is_full = wid < nfull
    is_tail = wid == nfull

    @pl.when(is_full)
    def _():
        pltpu.sync_copy(noise_hbm.at[pl.ds(wid * _NPW, _NPW)], nidx_v)
        pltpu.async_copy(emb_hbm.at[nidx_v], nrows_v, nsem)

    @pl.when(is_tail)
    def _():
        pltpu.sync_copy(noise_hbm.at[pl.ds(nfull * _NPW, 4)],
                        nidx_v.at[pl.ds(0, 4)])
        pltpu.async_copy(emb_hbm.at[nidx_v.at[pl.ds(0, 4)]],
                         nrows_v.at[pl.ds(0, 4)], nsem)

    for r in range(nrow):
        pltpu.make_async_copy(emb_hbm.at[idx_v.at[r]],
                              rows_v.at[pl.ds(r * _N, _N)], sem).wait()
        pltpu.async_copy(rows_v.at[pl.ds(r * _N, _N)],
                         rows_out.at[pl.ds(base + r * _N, _N)], wsem)

    @pl.when(is_full)
    def _():
        pltpu.make_async_copy(emb_hbm.at[nidx_v], nrows_v, nsem).wait()
        pltpu.async_copy(nrows_v, rows_out.at[pl.ds(_BN + wid * _NPW, _NPW)],
                         wsem)
        pltpu.make_async_copy(
            nrows_v, rows_out.at[pl.ds(_BN + wid * _NPW, _NPW)], wsem).wait()

    @pl.when(is_tail)
    def _():
        pltpu.make_async_copy(emb_hbm.at[nidx_v.at[pl.ds(0, 4)]],
                              nrows_v.at[pl.ds(0, 4)], nsem).wait()
        pltpu.async_copy(nrows_v.at[pl.ds(0, 4)],
                         rows_out.at[pl.ds(_BN + nfull * _NPW, 4)], wsem)
        pltpu.make_async_copy(
            nrows_v.at[pl.ds(0, 4)],
            rows_out.at[pl.ds(_BN + nfull * _NPW, 4)], wsem).wait()

    for r in range(nrow):
        pltpu.make_async_copy(rows_v.at[pl.ds(r * _N, _N)],
                              rows_out.at[pl.ds(base + r * _N, _N)],
                              wsem).wait()


@functools.cache
def _make_sc_gather():
  return pl.kernel(
    _sc_gather_body,
    out_type=jax.ShapeDtypeStruct((_BN + _NPAD, _D), jnp.float32),
    mesh=plsc.VectorSubcoreMesh(core_axis_name="c", subcore_axis_name="s",
                                num_cores=_NC, num_subcores=_NS),
    scratch_types=[
        pltpu.VMEM((_TPW // _N, _N), jnp.int32),
        pltpu.VMEM((_TPW, _D), jnp.float32),
        pltpu.VMEM((_NPW,), jnp.int32),
        pltpu.VMEM((_NPW, _D), jnp.float32),
        pltpu.SemaphoreType.DMA,
        pltpu.SemaphoreType.DMA,
        pltpu.SemaphoreType.DMA,
    ],
  )


def _tc_body(x_ref, rows_ref, tgt_ref, nz_ref, bias_ref, out_ref):
    x3 = x_ref[...]
    x2 = x3.reshape(_BN, _D)
    te3 = rows_ref[: _BN, :].reshape(_B, _N, _D)
    ne = rows_ref[_BN:, :]
    bias = bias_ref[...]
    tgt = tgt_ref[...]
    vid_t = lax.broadcasted_iota(jnp.int32, (_B, _N, _VOCAB), 2)
    bt = jnp.sum(jnp.where(vid_t == tgt[:, :, None], bias, 0.0), axis=2)
    nz = nz_ref[...].reshape(1, _K)
    vid_n = lax.broadcasted_iota(jnp.int32, (_VOCAB, _K), 0)
    onehot_n = jnp.where(vid_n == nz, 1.0, 0.0)
    bn = lax.dot_general(bias.reshape(1, _VOCAB), onehot_n,
                         (((1,), (0,)), ((), ())),
                         preferred_element_type=jnp.float32)

    t = jnp.sum(x3 * te3, axis=2) + bt - _LOGK
    s = lax.dot_general(x2, ne, (((1,), (1,)), ((), ())),
                        preferred_element_type=jnp.float32)[:, : _K]
    xn = s + bn - _LOGK
    sp_n = jnp.maximum(xn, 0.0) + jnp.log1p(jnp.exp(-jnp.abs(xn)))
    sp_t = jnp.maximum(-t, 0.0) + jnp.log1p(jnp.exp(-jnp.abs(t)))
    total = (jnp.sum(sp_n) + jnp.sum(sp_t)) * (1.0 / _BN)
    out_ref[0] = total


def kernel(target, input, emb, bias, noise_samples):
    rows = _make_sc_gather()(target, noise_samples.reshape(_K), emb)
    out = pl.pallas_call(
        _tc_body,
        out_shape=jax.ShapeDtypeStruct((1,), jnp.float32),
        out_specs=pl.BlockSpec(memory_space=pltpu.SMEM),
    )(input, rows, target, noise_samples.reshape(_K),
      bias.astype(jnp.float32))
    return out.reshape(())


# speedup vs baseline: 1.0250x; 1.0250x over previous
import functools
import math

import jax
import jax.numpy as jnp
from jax import lax
from jax.experimental import pallas as pl
from jax.experimental.pallas import tpu as pltpu
from jax.experimental.pallas import tpu_sc as plsc

_VOCAB = 1000
_D = 128
_B, _N = 64, 32
_K = 100
_BN = _B * _N
_LOGK = math.log(_K)

_NC, _NS = 2, 16
_NW = _NC * _NS
_TPW = _BN // _NW


def _sc_gather_body(tgt_hbm, emb_hbm, rows_out, idx_v, rows_v, sem, wsem):
    wid = lax.axis_index("s") * _NC + lax.axis_index("c")
    base = wid * _TPW
    nrow = _TPW // _N
    pltpu.sync_copy(tgt_hbm.at[pl.ds(wid * nrow, nrow), :], idx_v)
    for r in range(nrow):
        pltpu.async_copy(emb_hbm.at[idx_v.at[r]],
                         rows_v.at[pl.ds(r * _N, _N)], sem)
    for r in range(nrow):
        pltpu.make_async_copy(emb_hbm.at[idx_v.at[r]],
                              rows_v.at[pl.ds(r * _N, _N)], sem).wait()
        pltpu.async_copy(rows_v.at[pl.ds(r * _N, _N)],
                         rows_out.at[pl.ds(base + r * _N, _N)], wsem)
    for r in range(nrow):
        pltpu.make_async_copy(rows_v.at[pl.ds(r * _N, _N)],
                              rows_out.at[pl.ds(base + r * _N, _N)],
                              wsem).wait()


@functools.cache
def _make_sc_gather():
  return pl.kernel(
    _sc_gather_body,
    out_type=jax.ShapeDtypeStruct((_BN, _D), jnp.float32),
    mesh=plsc.VectorSubcoreMesh(core_axis_name="c", subcore_axis_name="s",
                                num_cores=_NC, num_subcores=_NS),
    scratch_types=[
        pltpu.VMEM((_TPW // _N, _N), jnp.int32),
        pltpu.VMEM((_TPW, _D), jnp.float32),
        pltpu.SemaphoreType.DMA,
        pltpu.SemaphoreType.DMA,
    ],
  )


def _tc_a_body(x_ref, nz_ref, bias_ref, emb_ref, tgt_ref, sn_ref, bt_ref):
    x2 = x_ref[...].reshape(_BN, _D)
    bias = bias_ref[...]
    nz = nz_ref[...].reshape(1, _K)
    vid_n = lax.broadcasted_iota(jnp.int32, (_VOCAB, _K), 0)
    onehot_n = jnp.where(vid_n == nz, 1.0, 0.0)
    ne = lax.dot_general(onehot_n, emb_ref[...],
                         (((0,), (0,)), ((), ())),
                         preferred_element_type=jnp.float32)
    bn = lax.dot_general(bias.reshape(1, _VOCAB), onehot_n,
                         (((1,), (0,)), ((), ())),
                         preferred_element_type=jnp.float32)
    s = lax.dot_general(x2, ne, (((1,), (1,)), ((), ())),
                        preferred_element_type=jnp.float32)
    xn = s + bn - _LOGK
    sp_n = jnp.maximum(xn, 0.0) + jnp.log1p(jnp.exp(-jnp.abs(xn)))
    sn_ref[0] = jnp.sum(sp_n)
    tgt = tgt_ref[...]
    vid_t = lax.broadcasted_iota(jnp.int32, (_B, _N, _VOCAB), 2)
    bt_ref[...] = jnp.sum(jnp.where(vid_t == tgt[:, :, None], bias, 0.0),
                          axis=2)


def _tc_b_body(x_ref, rows_ref, bt_ref, sn_ref, out_ref):
    x3 = x_ref[...]
    te3 = rows_ref[...].reshape(_B, _N, _D)
    t = jnp.sum(x3 * te3, axis=2) + bt_ref[...] - _LOGK
    sp_t = jnp.maximum(-t, 0.0) + jnp.log1p(jnp.exp(-jnp.abs(t)))
    out_ref[0] = (jnp.sum(sp_t) + sn_ref[0]) * (1.0 / _BN)


def kernel(target, input, emb, bias, noise_samples):
    rows = _make_sc_gather()(target, emb)
    sn, bt = pl.pallas_call(
        _tc_a_body,
        out_shape=(jax.ShapeDtypeStruct((1,), jnp.float32),
                   jax.ShapeDtypeStruct((_B, _N), jnp.float32)),
        out_specs=(pl.BlockSpec(memory_space=pltpu.SMEM),
                   pl.BlockSpec(memory_space=pltpu.VMEM)),
    )(input, noise_samples.reshape(_K), bias.astype(jnp.float32), emb,
      target)
    out = pl.pallas_call(
        _tc_b_body,
        out_shape=jax.ShapeDtypeStruct((1,), jnp.float32),
        in_specs=[pl.BlockSpec(memory_space=pltpu.VMEM),
                  pl.BlockSpec(memory_space=pltpu.VMEM),
                  pl.BlockSpec(memory_space=pltpu.VMEM),
                  pl.BlockSpec(memory_space=pltpu.SMEM)],
        out_specs=pl.BlockSpec(memory_space=pltpu.SMEM),
    )(input, rows, bt, sn)
    return out.reshape(())
